# baseline (speedup 1.0000x reference)
import functools

import jax
import jax.numpy as jnp
from jax import lax
from jax.experimental import pallas as pl
from jax.experimental.pallas import tpu as pltpu
from jax.experimental.pallas import tpu_sc as plsc

N_FEATURES = 26
VOCAB = 100000
EMBED_DIM = 64
BATCH = 16384

R = BATCH * N_FEATURES
NC, NS, L = 2, 16, 16
NW = NC * NS
ROWS_PER_W = R // NW
CHUNK = 512
GROUP = 128
N_GROUPS = CHUNK // GROUP
N_CHUNKS = ROWS_PER_W // CHUNK

_mesh = plsc.VectorSubcoreMesh(core_axis_name="c", subcore_axis_name="s")


@functools.partial(
    pl.kernel,
    out_type=jax.ShapeDtypeStruct((R, EMBED_DIM), jnp.float32),
    mesh=_mesh,
    scratch_types=[
        pltpu.VMEM((CHUNK,), jnp.int32),
        pltpu.VMEM((CHUNK, EMBED_DIM), jnp.float32),
        pltpu.SemaphoreType.DMA,
    ],
    compiler_params=pltpu.CompilerParams(use_tc_tiling_on_sc=False),
)
def _emb_gather(table_hbm, idx_hbm, out_hbm, idx_v, rows_v, sem):
    wid = lax.axis_index("s") * NC + lax.axis_index("c")
    wbase = wid * ROWS_PER_W
    iota = lax.iota(jnp.int32, L)

    def chunk_body(k, carry):
        base = wbase + k * CHUNK
        pltpu.sync_copy(idx_hbm.at[pl.ds(base, CHUNK)], idx_v)

        def vec_body(i, c):
            off = lax.rem(base + i * L + iota, N_FEATURES) * VOCAB
            idx_v[pl.ds(i * L, L)] = idx_v[pl.ds(i * L, L)] + off
            return c

        lax.fori_loop(0, CHUNK // L, vec_body, 0)

        copies = [
            pltpu.async_copy(
                table_hbm.at[idx_v.at[pl.ds(j * GROUP, GROUP)]],
                rows_v.at[pl.ds(j * GROUP, GROUP)],
                sem,
            )
            for j in range(N_GROUPS)
        ]
        for c in copies:
            c.wait()
        pltpu.sync_copy(rows_v, out_hbm.at[pl.ds(base, CHUNK)])
        return carry

    lax.fori_loop(0, N_CHUNKS, chunk_body, 0)


def kernel(indices, tables):
    flat_idx = indices.reshape(R)
    flat_tab = tables.reshape(N_FEATURES * VOCAB, EMBED_DIM)
    out = _emb_gather(flat_tab, flat_idx)
    return out.reshape(BATCH, N_FEATURES * EMBED_DIM)
